# Initial kernel scaffold; baseline (speedup 1.0000x reference)
import jax
import jax.numpy as jnp
from jax.experimental import pallas as pl


def kernel(u_feat, v_feat, params, inter_src, inter_dst, social_src, social_dst):
    raise NotImplementedError("write your pallas kernel here")


# speedup vs baseline: 1.3711x; 1.3711x over previous
import functools

import jax
import jax.numpy as jnp
from jax import lax
from jax.experimental import pallas as pl
from jax.experimental.pallas import tpu as pltpu
from jax.experimental.pallas import tpu_sc as plsc

_PREC = lax.Precision.HIGHEST

N = 4096
D_U = 256
D_V = 2304
D_OUT = 512
D_EMB = 64
E = 65536

_NW = 32
_ROWS = 16
_PASSES = N // (_ROWS * _NW)
_CHUNK = 2048


def _adj_body(isrc_hbm, idst_hbm, ssrc_hbm, sdst_hbm,
              a_inter_hbm, a_soc_hbm,
              ablk, src_v, dst_v):
    wid = lax.axis_index("s") * 2 + lax.axis_index("c")
    zeros16 = jnp.zeros((16,), jnp.float32)
    ones16 = jnp.ones((16,), jnp.float32)

    def do_graph(src_hbm, dst_hbm, a_hbm):
        def do_pass(p, carry):
            r0 = (p * _NW + wid) * _ROWS

            def zf(i, c):
                ablk[pl.ds(i * 16, 16)] = zeros16
                return c
            lax.fori_loop(0, _ROWS * N // 16, zf, 0)

            def chunk_body(ci, c):
                pltpu.sync_copy(src_hbm.at[pl.ds(ci * _CHUNK, _CHUNK)], src_v)
                pltpu.sync_copy(dst_hbm.at[pl.ds(ci * _CHUNK, _CHUNK)], dst_v)

                def ebody(i, cc):
                    s16 = src_v[pl.ds(i * 16, 16)]
                    d16 = dst_v[pl.ds(i * 16, 16)]
                    rel = d16 - r0
                    msk = (rel >= 0) & (rel < _ROWS)
                    relc = jnp.clip(rel, 0, _ROWS - 1)
                    plsc.addupdate_scatter(ablk, [relc * N + s16], ones16,
                                           mask=msk)
                    return cc
                lax.fori_loop(0, _CHUNK // 16, ebody, 0)
                return c
            lax.fori_loop(0, E // _CHUNK, chunk_body, 0)
            pltpu.sync_copy(ablk, a_hbm.at[pl.ds(r0 * N, _ROWS * N)])
            return carry
        lax.fori_loop(0, _PASSES, do_pass, 0)

    do_graph(isrc_hbm, idst_hbm, a_inter_hbm)
    do_graph(ssrc_hbm, sdst_hbm, a_soc_hbm)


def _build_adj(isrc, idst, ssrc, sdst):
    mesh = plsc.VectorSubcoreMesh(core_axis_name="c", subcore_axis_name="s")
    f = pl.kernel(
        _adj_body,
        out_type=(jax.ShapeDtypeStruct((N * N,), jnp.float32),
                  jax.ShapeDtypeStruct((N * N,), jnp.float32)),
        mesh=mesh,
        scratch_types=[
            pltpu.VMEM((_ROWS * N,), jnp.float32),
            pltpu.VMEM((_CHUNK,), jnp.int32),
            pltpu.VMEM((_CHUNK,), jnp.int32),
        ],
        compiler_params=pltpu.CompilerParams(needs_layout_passes=False),
    )
    ai, asoc = f(isrc, idst, ssrc, sdst)
    return ai.reshape(N, N), asoc.reshape(N, N)


_BM = 256


def _proj_body(vf, uf, Wiv, Wv, Wiu, Wu, bi, bu, bv, hv_o, us_o, u_o, v_o):
    vfb = vf[...]
    ufb = uf[...]
    hv_o[...] = jnp.dot(vfb, Wiv[...], preferred_element_type=jnp.float32, precision=_PREC)
    us_o[...] = jnp.dot(ufb, Wiu[...], preferred_element_type=jnp.float32, precision=_PREC) + bi[...]
    u_o[...] = jnp.dot(ufb, Wu[...], preferred_element_type=jnp.float32, precision=_PREC) + bu[...]
    v_o[...] = jnp.dot(vfb, Wv[...], preferred_element_type=jnp.float32, precision=_PREC) + bv[...]


def _proj(v_feat, u_feat, Wiv, Wv, Wiu, Wu, bi, bu, bv):
    grid = (N // _BM,)
    return pl.pallas_call(
        _proj_body,
        grid=grid,
        in_specs=[
            pl.BlockSpec((_BM, D_V), lambda i: (i, 0)),
            pl.BlockSpec((_BM, D_U), lambda i: (i, 0)),
            pl.BlockSpec((D_V, D_OUT), lambda i: (0, 0)),
            pl.BlockSpec((D_V, D_EMB), lambda i: (0, 0)),
            pl.BlockSpec((D_U, D_OUT), lambda i: (0, 0)),
            pl.BlockSpec((D_U, D_EMB), lambda i: (0, 0)),
            pl.BlockSpec((1, D_OUT), lambda i: (0, 0)),
            pl.BlockSpec((1, D_EMB), lambda i: (0, 0)),
            pl.BlockSpec((1, D_EMB), lambda i: (0, 0)),
        ],
        out_specs=[
            pl.BlockSpec((_BM, D_OUT), lambda i: (i, 0)),
            pl.BlockSpec((_BM, D_OUT), lambda i: (i, 0)),
            pl.BlockSpec((_BM, D_EMB), lambda i: (i, 0)),
            pl.BlockSpec((_BM, D_EMB), lambda i: (i, 0)),
        ],
        out_shape=[
            jax.ShapeDtypeStruct((N, D_OUT), jnp.float32),
            jax.ShapeDtypeStruct((N, D_OUT), jnp.float32),
            jax.ShapeDtypeStruct((N, D_EMB), jnp.float32),
            jax.ShapeDtypeStruct((N, D_EMB), jnp.float32),
        ],
    )(v_feat, u_feat, Wiv, Wv, Wiu, Wu, bi, bu, bv)


def _gcn_body(a, h, st, W1, b1, W2, b2, o1, o2):
    ab = a[...]
    msg = jnp.dot(ab, h[...], preferred_element_type=jnp.float32, precision=_PREC)
    deg = jnp.sum(ab, axis=1, keepdims=True)
    x = jnp.maximum(msg / (deg + 1.0) + st[...], 0.0)
    o1[...] = jnp.dot(x, W1[...], preferred_element_type=jnp.float32, precision=_PREC) + b1[...]
    o2[...] = jnp.dot(x, W2[...], preferred_element_type=jnp.float32, precision=_PREC) + b2[...]


def _gcn(A, h, st, W1, b1, W2, b2):
    d1 = W1.shape[1]
    d2 = W2.shape[1]
    grid = (N // _BM,)
    return pl.pallas_call(
        _gcn_body,
        grid=grid,
        in_specs=[
            pl.BlockSpec((_BM, N), lambda i: (i, 0)),
            pl.BlockSpec((N, D_OUT), lambda i: (0, 0)),
            pl.BlockSpec((_BM, D_OUT), lambda i: (i, 0)),
            pl.BlockSpec((D_OUT, d1), lambda i: (0, 0)),
            pl.BlockSpec((1, d1), lambda i: (0, 0)),
            pl.BlockSpec((D_OUT, d2), lambda i: (0, 0)),
            pl.BlockSpec((1, d2), lambda i: (0, 0)),
        ],
        out_specs=[
            pl.BlockSpec((_BM, d1), lambda i: (i, 0)),
            pl.BlockSpec((_BM, d2), lambda i: (i, 0)),
        ],
        out_shape=[
            jax.ShapeDtypeStruct((N, d1), jnp.float32),
            jax.ShapeDtypeStruct((N, d2), jnp.float32),
        ],
    )(A, h, st, W1, b1, W2, b2)


_BQ = 512


def _attn_body(q, k, v, Wq, bq, Wk, bk, Wv, bv, Wo, bo, o):
    qh = jnp.dot(q[...], Wq[...], preferred_element_type=jnp.float32, precision=_PREC) + bq[...]
    kh = jnp.dot(k[...], Wk[...], preferred_element_type=jnp.float32, precision=_PREC) + bk[...]
    vh = jnp.dot(v[...], Wv[...], preferred_element_type=jnp.float32, precision=_PREC) + bv[...]
    s = lax.dot_general(qh, kh, (((1,), (1,)), ((), ())),
                        preferred_element_type=jnp.float32, precision=_PREC) * 0.125
    s = s - jnp.max(s, axis=1, keepdims=True)
    e = jnp.exp(s)
    p = e / jnp.sum(e, axis=1, keepdims=True)
    o[...] = jnp.dot(jnp.dot(p, vh, preferred_element_type=jnp.float32, precision=_PREC),
                     Wo[...], preferred_element_type=jnp.float32, precision=_PREC) + bo[...]


def _attn(q, k, v, Wq, bq, Wk, bk, Wv, bv, Wo, bo):
    grid = (N // _BQ,)
    wspec = pl.BlockSpec((D_EMB, D_EMB), lambda i: (0, 0))
    bspec = pl.BlockSpec((1, D_EMB), lambda i: (0, 0))
    return pl.pallas_call(
        _attn_body,
        grid=grid,
        in_specs=[
            pl.BlockSpec((_BQ, D_EMB), lambda i: (i, 0)),
            pl.BlockSpec((N, D_EMB), lambda i: (0, 0)),
            pl.BlockSpec((N, D_EMB), lambda i: (0, 0)),
            wspec, bspec, wspec, bspec, wspec, bspec, wspec, bspec,
        ],
        out_specs=pl.BlockSpec((_BQ, D_EMB), lambda i: (i, 0)),
        out_shape=jax.ShapeDtypeStruct((N, D_EMB), jnp.float32),
    )(q, k, v, Wq, bq, Wk, bk, Wv, bv, Wo, bo)


def _gat_body(a, u2, v2, Wsrc, Wdst, asrc, adst, bg, o):
    ab = a[...]
    hsrc = jnp.dot(v2[...], Wsrc[...], preferred_element_type=jnp.float32, precision=_PREC)
    hdst = jnp.dot(u2[...], Wdst[...], preferred_element_type=jnp.float32, precision=_PREC)
    ssrc = lax.dot_general(asrc[...], hsrc, (((1,), (1,)), ((), ())),
                           preferred_element_type=jnp.float32, precision=_PREC)
    sdst = jnp.dot(hdst, adst[...], preferred_element_type=jnp.float32, precision=_PREC)
    z = sdst + ssrc
    e = jnp.where(z >= 0, z, 0.2 * z)
    mask = ab > 0.0
    m = jnp.max(jnp.where(mask, e, -1e30), axis=1, keepdims=True)
    w = ab * jnp.exp(jnp.minimum(e - m, 0.0))
    denom = jnp.sum(w, axis=1, keepdims=True)
    out = jnp.dot(w, hsrc, preferred_element_type=jnp.float32, precision=_PREC)
    o[...] = out / (denom + 1e-9) + bg[...]


def _gat(A, u2, v2, Wsrc, Wdst, asrc, adst, bg):
    grid = (N // _BM,)
    return pl.pallas_call(
        _gat_body,
        grid=grid,
        in_specs=[
            pl.BlockSpec((_BM, N), lambda i: (i, 0)),
            pl.BlockSpec((_BM, 2 * D_EMB), lambda i: (i, 0)),
            pl.BlockSpec((N, 2 * D_EMB), lambda i: (0, 0)),
            pl.BlockSpec((2 * D_EMB, D_EMB), lambda i: (0, 0)),
            pl.BlockSpec((2 * D_EMB, D_EMB), lambda i: (0, 0)),
            pl.BlockSpec((1, D_EMB), lambda i: (0, 0)),
            pl.BlockSpec((D_EMB, 1), lambda i: (0, 0)),
            pl.BlockSpec((1, D_EMB), lambda i: (0, 0)),
        ],
        out_specs=pl.BlockSpec((_BM, D_EMB), lambda i: (i, 0)),
        out_shape=jax.ShapeDtypeStruct((N, D_EMB), jnp.float32),
    )(A, u2, v2, Wsrc, Wdst, asrc, adst, bg)


def kernel(u_feat, v_feat, params, inter_src, inter_dst, social_src, social_dst):
    p = params
    isrc = inter_src.astype(jnp.int32)
    idst = inter_dst.astype(jnp.int32)
    ssrc = social_src.astype(jnp.int32)
    sdst = social_dst.astype(jnp.int32)

    A_inter, A_soc = _build_adj(isrc, idst, ssrc, sdst)

    row = lambda b: b.reshape(1, -1)
    h_v, u_self, u, v = _proj(
        v_feat, u_feat, p['W_inter_v'], p['W_v'], p['W_inter_u'], p['W_u'],
        row(p['b_inter']), row(p['b_u']), row(p['b_v']))

    zero_row = jnp.zeros((1, D_OUT), jnp.float32)
    h_soc, xs = _gcn(A_inter, h_v, u_self,
                     p['W_soc_nbr'], zero_row,
                     p['W_soc_self'], row(p['b_soc']))
    Xe, _ = _gcn(A_soc, h_soc, xs,
                 p['W_x'], row(p['b_x']),
                 p['W_x'], row(p['b_x']))

    def attn(q, k, v_, name):
        return _attn(q, k, v_,
                     p[name + '_Wq'], row(p[name + '_bq']),
                     p[name + '_Wk'], row(p[name + '_bk']),
                     p[name + '_Wv'], row(p[name + '_bv']),
                     p[name + '_Wo'], row(p[name + '_bo']))

    f_uus = attn(u, Xe, Xe, 'hur')
    e_uv = attn(v, f_uus, f_uus, 'uvr')
    e_vu = attn(f_uus, v, v, 'vur')

    u2 = jnp.concatenate([u, e_vu], axis=1)
    v2 = jnp.concatenate([v, e_uv], axis=1)

    return _gat(A_inter, u2, v2,
                p['W_gat_src'], p['W_gat_dst'],
                p['a_src'].reshape(1, D_EMB), p['a_dst'].reshape(D_EMB, 1),
                row(p['b_gat']))


# speedup vs baseline: 1.5368x; 1.1208x over previous
import functools

import jax
import jax.numpy as jnp
from jax import lax
from jax.experimental import pallas as pl
from jax.experimental.pallas import tpu as pltpu
from jax.experimental.pallas import tpu_sc as plsc

_PREC = lax.Precision.HIGHEST

N = 4096
D_U = 256
D_V = 2304
D_OUT = 512
D_EMB = 64
E = 65536

_NW = 32
_ROWS = 16
_PASSES = N // (_ROWS * _NW)
_CHUNK = 2048


def _adj_body(isrc_hbm, idst_hbm, ssrc_hbm, sdst_hbm, zeros_hbm,
              a_inter_hbm, a_soc_hbm,
              ablk, src_v, dst_v):
    wid = lax.axis_index("s") * 2 + lax.axis_index("c")
    ones16 = jnp.ones((16,), jnp.float32)

    def do_graph(src_hbm, dst_hbm, a_hbm):
        def do_pass(p, carry):
            r0 = (p * _NW + wid) * _ROWS
            pltpu.sync_copy(zeros_hbm, ablk)

            def chunk_body(ci, c):
                pltpu.sync_copy(src_hbm.at[pl.ds(ci * _CHUNK, _CHUNK)], src_v)
                pltpu.sync_copy(dst_hbm.at[pl.ds(ci * _CHUNK, _CHUNK)], dst_v)

                def ebody(i, cc):
                    s16 = src_v[pl.ds(i * 16, 16)]
                    d16 = dst_v[pl.ds(i * 16, 16)]
                    rel = d16 - r0
                    msk = plsc.bitcast(rel, jnp.uint32) < jnp.uint32(_ROWS)
                    plsc.addupdate_scatter(ablk, [rel * N + s16], ones16,
                                           mask=msk)
                    return cc
                lax.fori_loop(0, _CHUNK // 16, ebody, 0)
                return c
            lax.fori_loop(0, E // _CHUNK, chunk_body, 0)
            pltpu.sync_copy(ablk, a_hbm.at[pl.ds(r0 * N, _ROWS * N)])
            return carry
        lax.fori_loop(0, _PASSES, do_pass, 0)

    do_graph(isrc_hbm, idst_hbm, a_inter_hbm)
    do_graph(ssrc_hbm, sdst_hbm, a_soc_hbm)


def _build_adj(isrc, idst, ssrc, sdst):
    mesh = plsc.VectorSubcoreMesh(core_axis_name="c", subcore_axis_name="s")
    f = pl.kernel(
        _adj_body,
        out_type=(jax.ShapeDtypeStruct((N * N,), jnp.float32),
                  jax.ShapeDtypeStruct((N * N,), jnp.float32)),
        mesh=mesh,
        scratch_types=[
            pltpu.VMEM((_ROWS * N,), jnp.float32),
            pltpu.VMEM((_CHUNK,), jnp.int32),
            pltpu.VMEM((_CHUNK,), jnp.int32),
        ],
        compiler_params=pltpu.CompilerParams(needs_layout_passes=False),
    )
    zeros_blk = jnp.zeros((_ROWS * N,), jnp.float32)
    ai, asoc = f(isrc, idst, ssrc, sdst, zeros_blk)
    return ai.reshape(N, N), asoc.reshape(N, N)


_BM = 256


def _proj_body(vf, uf, Wiv, Wv, Wiu, Wu, bi, bu, bv, hv_o, us_o, u_o, v_o):
    vfb = vf[...]
    ufb = uf[...]
    hv_o[...] = jnp.dot(vfb, Wiv[...], preferred_element_type=jnp.float32, precision=_PREC)
    us_o[...] = jnp.dot(ufb, Wiu[...], preferred_element_type=jnp.float32, precision=_PREC) + bi[...]
    u_o[...] = jnp.dot(ufb, Wu[...], preferred_element_type=jnp.float32, precision=_PREC) + bu[...]
    v_o[...] = jnp.dot(vfb, Wv[...], preferred_element_type=jnp.float32, precision=_PREC) + bv[...]


def _proj(v_feat, u_feat, Wiv, Wv, Wiu, Wu, bi, bu, bv):
    grid = (N // _BM,)
    return pl.pallas_call(
        _proj_body,
        grid=grid,
        in_specs=[
            pl.BlockSpec((_BM, D_V), lambda i: (i, 0)),
            pl.BlockSpec((_BM, D_U), lambda i: (i, 0)),
            pl.BlockSpec((D_V, D_OUT), lambda i: (0, 0)),
            pl.BlockSpec((D_V, D_EMB), lambda i: (0, 0)),
            pl.BlockSpec((D_U, D_OUT), lambda i: (0, 0)),
            pl.BlockSpec((D_U, D_EMB), lambda i: (0, 0)),
            pl.BlockSpec((1, D_OUT), lambda i: (0, 0)),
            pl.BlockSpec((1, D_EMB), lambda i: (0, 0)),
            pl.BlockSpec((1, D_EMB), lambda i: (0, 0)),
        ],
        out_specs=[
            pl.BlockSpec((_BM, D_OUT), lambda i: (i, 0)),
            pl.BlockSpec((_BM, D_OUT), lambda i: (i, 0)),
            pl.BlockSpec((_BM, D_EMB), lambda i: (i, 0)),
            pl.BlockSpec((_BM, D_EMB), lambda i: (i, 0)),
        ],
        out_shape=[
            jax.ShapeDtypeStruct((N, D_OUT), jnp.float32),
            jax.ShapeDtypeStruct((N, D_OUT), jnp.float32),
            jax.ShapeDtypeStruct((N, D_EMB), jnp.float32),
            jax.ShapeDtypeStruct((N, D_EMB), jnp.float32),
        ],
    )(v_feat, u_feat, Wiv, Wv, Wiu, Wu, bi, bu, bv)


def _gcn_body(a, h, st, W1, b1, W2, b2, o1, o2):
    ab = a[...]
    msg = jnp.dot(ab, h[...], preferred_element_type=jnp.float32, precision=_PREC)
    deg = jnp.sum(ab, axis=1, keepdims=True)
    x = jnp.maximum(msg / (deg + 1.0) + st[...], 0.0)
    o1[...] = jnp.dot(x, W1[...], preferred_element_type=jnp.float32, precision=_PREC) + b1[...]
    o2[...] = jnp.dot(x, W2[...], preferred_element_type=jnp.float32, precision=_PREC) + b2[...]


def _gcn(A, h, st, W1, b1, W2, b2):
    d1 = W1.shape[1]
    d2 = W2.shape[1]
    grid = (N // _BM,)
    return pl.pallas_call(
        _gcn_body,
        grid=grid,
        in_specs=[
            pl.BlockSpec((_BM, N), lambda i: (i, 0)),
            pl.BlockSpec((N, D_OUT), lambda i: (0, 0)),
            pl.BlockSpec((_BM, D_OUT), lambda i: (i, 0)),
            pl.BlockSpec((D_OUT, d1), lambda i: (0, 0)),
            pl.BlockSpec((1, d1), lambda i: (0, 0)),
            pl.BlockSpec((D_OUT, d2), lambda i: (0, 0)),
            pl.BlockSpec((1, d2), lambda i: (0, 0)),
        ],
        out_specs=[
            pl.BlockSpec((_BM, d1), lambda i: (i, 0)),
            pl.BlockSpec((_BM, d2), lambda i: (i, 0)),
        ],
        out_shape=[
            jax.ShapeDtypeStruct((N, d1), jnp.float32),
            jax.ShapeDtypeStruct((N, d2), jnp.float32),
        ],
    )(A, h, st, W1, b1, W2, b2)


_BQ = 512


def _attn_body(q, k, v, Wq, bq, Wk, bk, Wv, bv, Wo, bo, o):
    qh = jnp.dot(q[...], Wq[...], preferred_element_type=jnp.float32, precision=_PREC) + bq[...]
    kh = jnp.dot(k[...], Wk[...], preferred_element_type=jnp.float32, precision=_PREC) + bk[...]
    vh = jnp.dot(v[...], Wv[...], preferred_element_type=jnp.float32, precision=_PREC) + bv[...]
    s = lax.dot_general(qh, kh, (((1,), (1,)), ((), ())),
                        preferred_element_type=jnp.float32, precision=_PREC) * 0.125
    s = s - jnp.max(s, axis=1, keepdims=True)
    e = jnp.exp(s)
    p = e / jnp.sum(e, axis=1, keepdims=True)
    o[...] = jnp.dot(jnp.dot(p, vh, preferred_element_type=jnp.float32, precision=_PREC),
                     Wo[...], preferred_element_type=jnp.float32, precision=_PREC) + bo[...]


def _attn(q, k, v, Wq, bq, Wk, bk, Wv, bv, Wo, bo):
    grid = (N // _BQ,)
    wspec = pl.BlockSpec((D_EMB, D_EMB), lambda i: (0, 0))
    bspec = pl.BlockSpec((1, D_EMB), lambda i: (0, 0))
    return pl.pallas_call(
        _attn_body,
        grid=grid,
        in_specs=[
            pl.BlockSpec((_BQ, D_EMB), lambda i: (i, 0)),
            pl.BlockSpec((N, D_EMB), lambda i: (0, 0)),
            pl.BlockSpec((N, D_EMB), lambda i: (0, 0)),
            wspec, bspec, wspec, bspec, wspec, bspec, wspec, bspec,
        ],
        out_specs=pl.BlockSpec((_BQ, D_EMB), lambda i: (i, 0)),
        out_shape=jax.ShapeDtypeStruct((N, D_EMB), jnp.float32),
    )(q, k, v, Wq, bq, Wk, bk, Wv, bv, Wo, bo)


def _gat_body(a, u2, v2, Wsrc, Wdst, asrc, adst, bg, o):
    ab = a[...]
    hsrc = jnp.dot(v2[...], Wsrc[...], preferred_element_type=jnp.float32, precision=_PREC)
    hdst = jnp.dot(u2[...], Wdst[...], preferred_element_type=jnp.float32, precision=_PREC)
    ssrc = lax.dot_general(asrc[...], hsrc, (((1,), (1,)), ((), ())),
                           preferred_element_type=jnp.float32, precision=_PREC)
    sdst = jnp.dot(hdst, adst[...], preferred_element_type=jnp.float32, precision=_PREC)
    z = sdst + ssrc
    e = jnp.where(z >= 0, z, 0.2 * z)
    mask = ab > 0.0
    m = jnp.max(jnp.where(mask, e, -1e30), axis=1, keepdims=True)
    w = ab * jnp.exp(jnp.minimum(e - m, 0.0))
    denom = jnp.sum(w, axis=1, keepdims=True)
    out = jnp.dot(w, hsrc, preferred_element_type=jnp.float32, precision=_PREC)
    o[...] = out / (denom + 1e-9) + bg[...]


def _gat(A, u2, v2, Wsrc, Wdst, asrc, adst, bg):
    grid = (N // _BM,)
    return pl.pallas_call(
        _gat_body,
        grid=grid,
        in_specs=[
            pl.BlockSpec((_BM, N), lambda i: (i, 0)),
            pl.BlockSpec((_BM, 2 * D_EMB), lambda i: (i, 0)),
            pl.BlockSpec((N, 2 * D_EMB), lambda i: (0, 0)),
            pl.BlockSpec((2 * D_EMB, D_EMB), lambda i: (0, 0)),
            pl.BlockSpec((2 * D_EMB, D_EMB), lambda i: (0, 0)),
            pl.BlockSpec((1, D_EMB), lambda i: (0, 0)),
            pl.BlockSpec((D_EMB, 1), lambda i: (0, 0)),
            pl.BlockSpec((1, D_EMB), lambda i: (0, 0)),
        ],
        out_specs=pl.BlockSpec((_BM, D_EMB), lambda i: (i, 0)),
        out_shape=jax.ShapeDtypeStruct((N, D_EMB), jnp.float32),
    )(A, u2, v2, Wsrc, Wdst, asrc, adst, bg)


def kernel(u_feat, v_feat, params, inter_src, inter_dst, social_src, social_dst):
    p = params
    isrc = inter_src.astype(jnp.int32)
    idst = inter_dst.astype(jnp.int32)
    ssrc = social_src.astype(jnp.int32)
    sdst = social_dst.astype(jnp.int32)

    A_inter, A_soc = _build_adj(isrc, idst, ssrc, sdst)

    row = lambda b: b.reshape(1, -1)
    h_v, u_self, u, v = _proj(
        v_feat, u_feat, p['W_inter_v'], p['W_v'], p['W_inter_u'], p['W_u'],
        row(p['b_inter']), row(p['b_u']), row(p['b_v']))

    zero_row = jnp.zeros((1, D_OUT), jnp.float32)
    h_soc, xs = _gcn(A_inter, h_v, u_self,
                     p['W_soc_nbr'], zero_row,
                     p['W_soc_self'], row(p['b_soc']))
    Xe, _ = _gcn(A_soc, h_soc, xs,
                 p['W_x'], row(p['b_x']),
                 p['W_x'], row(p['b_x']))

    def attn(q, k, v_, name):
        return _attn(q, k, v_,
                     p[name + '_Wq'], row(p[name + '_bq']),
                     p[name + '_Wk'], row(p[name + '_bk']),
                     p[name + '_Wv'], row(p[name + '_bv']),
                     p[name + '_Wo'], row(p[name + '_bo']))

    f_uus = attn(u, Xe, Xe, 'hur')
    e_uv = attn(v, f_uus, f_uus, 'uvr')
    e_vu = attn(f_uus, v, v, 'vur')

    u2 = jnp.concatenate([u, e_vu], axis=1)
    v2 = jnp.concatenate([v, e_uv], axis=1)

    return _gat(A_inter, u2, v2,
                p['W_gat_src'], p['W_gat_dst'],
                p['a_src'].reshape(1, D_EMB), p['a_dst'].reshape(D_EMB, 1),
                row(p['b_gat']))


# speedup vs baseline: 1.8089x; 1.1771x over previous
import functools

import jax
import jax.numpy as jnp
from jax import lax
from jax.experimental import pallas as pl
from jax.experimental.pallas import tpu as pltpu
from jax.experimental.pallas import tpu_sc as plsc

_PREC = lax.Precision.HIGHEST

N = 4096
D_U = 256
D_V = 2304
D_OUT = 512
D_EMB = 64
E = 65536

_NW = 32
_ROWS = 16
_PASSES = N // (_ROWS * _NW)
_CHUNK = 2048


def _adj_body(isrc_hbm, idst_hbm, ssrc_hbm, sdst_hbm, zeros_hbm,
              a_inter_hbm, a_soc_hbm,
              ablk, src_v, dst_v):
    wid = lax.axis_index("s") * 2 + lax.axis_index("c")
    ones16 = jnp.ones((16,), jnp.float32)

    def do_graph(src_hbm, dst_hbm, a_hbm):
        def do_pass(p, carry):
            r0 = (p * _NW + wid) * _ROWS
            pltpu.sync_copy(zeros_hbm, ablk)

            def chunk_body(ci, c):
                pltpu.sync_copy(src_hbm.at[pl.ds(ci * _CHUNK, _CHUNK)], src_v)
                pltpu.sync_copy(dst_hbm.at[pl.ds(ci * _CHUNK, _CHUNK)], dst_v)

                @plsc.parallel_loop(0, _CHUNK, 16, unroll=8)
                def ebody(i):
                    s16 = src_v[pl.ds(i, 16)]
                    d16 = dst_v[pl.ds(i, 16)]
                    rel = d16 - r0
                    msk = plsc.bitcast(rel, jnp.uint32) < jnp.uint32(_ROWS)
                    plsc.addupdate_scatter(ablk, [rel * N + s16], ones16,
                                           mask=msk)
                return c
            lax.fori_loop(0, E // _CHUNK, chunk_body, 0)
            pltpu.sync_copy(ablk, a_hbm.at[pl.ds(r0 * N, _ROWS * N)])
            return carry
        lax.fori_loop(0, _PASSES, do_pass, 0)

    do_graph(isrc_hbm, idst_hbm, a_inter_hbm)
    do_graph(ssrc_hbm, sdst_hbm, a_soc_hbm)


def _build_adj(isrc, idst, ssrc, sdst):
    mesh = plsc.VectorSubcoreMesh(core_axis_name="c", subcore_axis_name="s")
    f = pl.kernel(
        _adj_body,
        out_type=(jax.ShapeDtypeStruct((N * N,), jnp.float32),
                  jax.ShapeDtypeStruct((N * N,), jnp.float32)),
        mesh=mesh,
        scratch_types=[
            pltpu.VMEM((_ROWS * N,), jnp.float32),
            pltpu.VMEM((_CHUNK,), jnp.int32),
            pltpu.VMEM((_CHUNK,), jnp.int32),
        ],
        compiler_params=pltpu.CompilerParams(needs_layout_passes=False),
    )
    zeros_blk = jnp.zeros((_ROWS * N,), jnp.float32)
    ai, asoc = f(isrc, idst, ssrc, sdst, zeros_blk)
    return ai.reshape(N, N), asoc.reshape(N, N)


_BM = 256


def _proj_body(vf, uf, Wiv, Wv, Wiu, Wu, bi, bu, bv, hv_o, us_o, u_o, v_o):
    vfb = vf[...]
    ufb = uf[...]
    hv_o[...] = jnp.dot(vfb, Wiv[...], preferred_element_type=jnp.float32, precision=_PREC)
    us_o[...] = jnp.dot(ufb, Wiu[...], preferred_element_type=jnp.float32, precision=_PREC) + bi[...]
    u_o[...] = jnp.dot(ufb, Wu[...], preferred_element_type=jnp.float32, precision=_PREC) + bu[...]
    v_o[...] = jnp.dot(vfb, Wv[...], preferred_element_type=jnp.float32, precision=_PREC) + bv[...]


def _proj(v_feat, u_feat, Wiv, Wv, Wiu, Wu, bi, bu, bv):
    grid = (N // _BM,)
    return pl.pallas_call(
        _proj_body,
        grid=grid,
        in_specs=[
            pl.BlockSpec((_BM, D_V), lambda i: (i, 0)),
            pl.BlockSpec((_BM, D_U), lambda i: (i, 0)),
            pl.BlockSpec((D_V, D_OUT), lambda i: (0, 0)),
            pl.BlockSpec((D_V, D_EMB), lambda i: (0, 0)),
            pl.BlockSpec((D_U, D_OUT), lambda i: (0, 0)),
            pl.BlockSpec((D_U, D_EMB), lambda i: (0, 0)),
            pl.BlockSpec((1, D_OUT), lambda i: (0, 0)),
            pl.BlockSpec((1, D_EMB), lambda i: (0, 0)),
            pl.BlockSpec((1, D_EMB), lambda i: (0, 0)),
        ],
        out_specs=[
            pl.BlockSpec((_BM, D_OUT), lambda i: (i, 0)),
            pl.BlockSpec((_BM, D_OUT), lambda i: (i, 0)),
            pl.BlockSpec((_BM, D_EMB), lambda i: (i, 0)),
            pl.BlockSpec((_BM, D_EMB), lambda i: (i, 0)),
        ],
        out_shape=[
            jax.ShapeDtypeStruct((N, D_OUT), jnp.float32),
            jax.ShapeDtypeStruct((N, D_OUT), jnp.float32),
            jax.ShapeDtypeStruct((N, D_EMB), jnp.float32),
            jax.ShapeDtypeStruct((N, D_EMB), jnp.float32),
        ],
    )(v_feat, u_feat, Wiv, Wv, Wiu, Wu, bi, bu, bv)


def _gcn_body(a, h, st, W1, b1, W2, b2, o1, o2):
    ab = a[...]
    msg = jnp.dot(ab, h[...], preferred_element_type=jnp.float32, precision=_PREC)
    deg = jnp.sum(ab, axis=1, keepdims=True)
    x = jnp.maximum(msg / (deg + 1.0) + st[...], 0.0)
    o1[...] = jnp.dot(x, W1[...], preferred_element_type=jnp.float32, precision=_PREC) + b1[...]
    o2[...] = jnp.dot(x, W2[...], preferred_element_type=jnp.float32, precision=_PREC) + b2[...]


def _gcn(A, h, st, W1, b1, W2, b2):
    d1 = W1.shape[1]
    d2 = W2.shape[1]
    grid = (N // _BM,)
    return pl.pallas_call(
        _gcn_body,
        grid=grid,
        in_specs=[
            pl.BlockSpec((_BM, N), lambda i: (i, 0)),
            pl.BlockSpec((N, D_OUT), lambda i: (0, 0)),
            pl.BlockSpec((_BM, D_OUT), lambda i: (i, 0)),
            pl.BlockSpec((D_OUT, d1), lambda i: (0, 0)),
            pl.BlockSpec((1, d1), lambda i: (0, 0)),
            pl.BlockSpec((D_OUT, d2), lambda i: (0, 0)),
            pl.BlockSpec((1, d2), lambda i: (0, 0)),
        ],
        out_specs=[
            pl.BlockSpec((_BM, d1), lambda i: (i, 0)),
            pl.BlockSpec((_BM, d2), lambda i: (i, 0)),
        ],
        out_shape=[
            jax.ShapeDtypeStruct((N, d1), jnp.float32),
            jax.ShapeDtypeStruct((N, d2), jnp.float32),
        ],
    )(A, h, st, W1, b1, W2, b2)


_BQ = 512


def _attn_body(q, k, v, Wq, bq, Wk, bk, Wv, bv, Wo, bo, o):
    qh = jnp.dot(q[...], Wq[...], preferred_element_type=jnp.float32, precision=_PREC) + bq[...]
    kh = jnp.dot(k[...], Wk[...], preferred_element_type=jnp.float32, precision=_PREC) + bk[...]
    vh = jnp.dot(v[...], Wv[...], preferred_element_type=jnp.float32, precision=_PREC) + bv[...]
    s = lax.dot_general(qh, kh, (((1,), (1,)), ((), ())),
                        preferred_element_type=jnp.float32, precision=_PREC) * 0.125
    s = s - jnp.max(s, axis=1, keepdims=True)
    e = jnp.exp(s)
    p = e / jnp.sum(e, axis=1, keepdims=True)
    o[...] = jnp.dot(jnp.dot(p, vh, preferred_element_type=jnp.float32, precision=_PREC),
                     Wo[...], preferred_element_type=jnp.float32, precision=_PREC) + bo[...]


def _attn(q, k, v, Wq, bq, Wk, bk, Wv, bv, Wo, bo):
    grid = (N // _BQ,)
    wspec = pl.BlockSpec((D_EMB, D_EMB), lambda i: (0, 0))
    bspec = pl.BlockSpec((1, D_EMB), lambda i: (0, 0))
    return pl.pallas_call(
        _attn_body,
        grid=grid,
        in_specs=[
            pl.BlockSpec((_BQ, D_EMB), lambda i: (i, 0)),
            pl.BlockSpec((N, D_EMB), lambda i: (0, 0)),
            pl.BlockSpec((N, D_EMB), lambda i: (0, 0)),
            wspec, bspec, wspec, bspec, wspec, bspec, wspec, bspec,
        ],
        out_specs=pl.BlockSpec((_BQ, D_EMB), lambda i: (i, 0)),
        out_shape=jax.ShapeDtypeStruct((N, D_EMB), jnp.float32),
    )(q, k, v, Wq, bq, Wk, bk, Wv, bv, Wo, bo)


def _gat_body(a, u2, v2, Wsrc, Wdst, asrc, adst, bg, o):
    ab = a[...]
    hsrc = jnp.dot(v2[...], Wsrc[...], preferred_element_type=jnp.float32, precision=_PREC)
    hdst = jnp.dot(u2[...], Wdst[...], preferred_element_type=jnp.float32, precision=_PREC)
    ssrc = lax.dot_general(asrc[...], hsrc, (((1,), (1,)), ((), ())),
                           preferred_element_type=jnp.float32, precision=_PREC)
    sdst = jnp.dot(hdst, adst[...], preferred_element_type=jnp.float32, precision=_PREC)
    z = sdst + ssrc
    e = jnp.where(z >= 0, z, 0.2 * z)
    mask = ab > 0.0
    m = jnp.max(jnp.where(mask, e, -1e30), axis=1, keepdims=True)
    w = ab * jnp.exp(jnp.minimum(e - m, 0.0))
    denom = jnp.sum(w, axis=1, keepdims=True)
    out = jnp.dot(w, hsrc, preferred_element_type=jnp.float32, precision=_PREC)
    o[...] = out / (denom + 1e-9) + bg[...]


def _gat(A, u2, v2, Wsrc, Wdst, asrc, adst, bg):
    grid = (N // _BM,)
    return pl.pallas_call(
        _gat_body,
        grid=grid,
        in_specs=[
            pl.BlockSpec((_BM, N), lambda i: (i, 0)),
            pl.BlockSpec((_BM, 2 * D_EMB), lambda i: (i, 0)),
            pl.BlockSpec((N, 2 * D_EMB), lambda i: (0, 0)),
            pl.BlockSpec((2 * D_EMB, D_EMB), lambda i: (0, 0)),
            pl.BlockSpec((2 * D_EMB, D_EMB), lambda i: (0, 0)),
            pl.BlockSpec((1, D_EMB), lambda i: (0, 0)),
            pl.BlockSpec((D_EMB, 1), lambda i: (0, 0)),
            pl.BlockSpec((1, D_EMB), lambda i: (0, 0)),
        ],
        out_specs=pl.BlockSpec((_BM, D_EMB), lambda i: (i, 0)),
        out_shape=jax.ShapeDtypeStruct((N, D_EMB), jnp.float32),
    )(A, u2, v2, Wsrc, Wdst, asrc, adst, bg)


def kernel(u_feat, v_feat, params, inter_src, inter_dst, social_src, social_dst):
    p = params
    isrc = inter_src.astype(jnp.int32)
    idst = inter_dst.astype(jnp.int32)
    ssrc = social_src.astype(jnp.int32)
    sdst = social_dst.astype(jnp.int32)

    A_inter, A_soc = _build_adj(isrc, idst, ssrc, sdst)

    row = lambda b: b.reshape(1, -1)
    h_v, u_self, u, v = _proj(
        v_feat, u_feat, p['W_inter_v'], p['W_v'], p['W_inter_u'], p['W_u'],
        row(p['b_inter']), row(p['b_u']), row(p['b_v']))

    zero_row = jnp.zeros((1, D_OUT), jnp.float32)
    h_soc, xs = _gcn(A_inter, h_v, u_self,
                     p['W_soc_nbr'], zero_row,
                     p['W_soc_self'], row(p['b_soc']))
    Xe, _ = _gcn(A_soc, h_soc, xs,
                 p['W_x'], row(p['b_x']),
                 p['W_x'], row(p['b_x']))

    def attn(q, k, v_, name):
        return _attn(q, k, v_,
                     p[name + '_Wq'], row(p[name + '_bq']),
                     p[name + '_Wk'], row(p[name + '_bk']),
                     p[name + '_Wv'], row(p[name + '_bv']),
                     p[name + '_Wo'], row(p[name + '_bo']))

    f_uus = attn(u, Xe, Xe, 'hur')
    e_uv = attn(v, f_uus, f_uus, 'uvr')
    e_vu = attn(f_uus, v, v, 'vur')

    u2 = jnp.concatenate([u, e_vu], axis=1)
    v2 = jnp.concatenate([v, e_uv], axis=1)

    return _gat(A_inter, u2, v2,
                p['W_gat_src'], p['W_gat_dst'],
                p['a_src'].reshape(1, D_EMB), p['a_dst'].reshape(D_EMB, 1),
                row(p['b_gat']))


# speedup vs baseline: 1.9750x; 1.0918x over previous
import functools

import jax
import jax.numpy as jnp
from jax import lax
from jax.experimental import pallas as pl
from jax.experimental.pallas import tpu as pltpu
from jax.experimental.pallas import tpu_sc as plsc

_PREC = lax.Precision.HIGHEST

N = 4096
D_U = 256
D_V = 2304
D_OUT = 512
D_EMB = 64
E = 65536

_NW = 32
_ROWS = 16
_PASSES = N // (_ROWS * _NW)
_CHUNK = 2048


def _adj_body(src_hbm, dst_hbm, zeros_hbm, a_hbm, ablk, src_v, dst_v):
    wid = lax.axis_index("s") * 2 + lax.axis_index("c")
    ones16 = jnp.ones((16,), jnp.float32)

    if True:
        def do_pass(p, carry):
            r0 = (p * _NW + wid) * _ROWS
            pltpu.sync_copy(zeros_hbm, ablk)

            def chunk_body(ci, c):
                pltpu.sync_copy(src_hbm.at[pl.ds(ci * _CHUNK, _CHUNK)], src_v)
                pltpu.sync_copy(dst_hbm.at[pl.ds(ci * _CHUNK, _CHUNK)], dst_v)

                @plsc.parallel_loop(0, _CHUNK, 16, unroll=8)
                def ebody(i):
                    s16 = src_v[pl.ds(i, 16)]
                    d16 = dst_v[pl.ds(i, 16)]
                    rel = d16 - r0
                    msk = plsc.bitcast(rel, jnp.uint32) < jnp.uint32(_ROWS)
                    plsc.addupdate_scatter(ablk, [rel * N + s16], ones16,
                                           mask=msk)
                return c
            lax.fori_loop(0, E // _CHUNK, chunk_body, 0)
            pltpu.sync_copy(ablk, a_hbm.at[pl.ds(r0 * N, _ROWS * N)])
            return carry
        lax.fori_loop(0, _PASSES, do_pass, 0)


def _build_adj_one(src, dst, zeros_blk):
    mesh = plsc.VectorSubcoreMesh(core_axis_name="c", subcore_axis_name="s")
    f = pl.kernel(
        _adj_body,
        out_type=jax.ShapeDtypeStruct((N * N,), jnp.float32),
        mesh=mesh,
        scratch_types=[
            pltpu.VMEM((_ROWS * N,), jnp.float32),
            pltpu.VMEM((_CHUNK,), jnp.int32),
            pltpu.VMEM((_CHUNK,), jnp.int32),
        ],
        compiler_params=pltpu.CompilerParams(needs_layout_passes=False),
    )
    return f(src, dst, zeros_blk).reshape(N, N)


def _build_adj(isrc, idst, ssrc, sdst):
    zeros_blk = jnp.zeros((_ROWS * N,), jnp.float32)
    ai = _build_adj_one(isrc, idst, zeros_blk)
    asoc = _build_adj_one(ssrc, sdst, zeros_blk)
    return ai, asoc


_BM = 256


def _proj_body(vf, uf, Wiv, Wv, Wiu, Wu, bi, bu, bv, hv_o, us_o, u_o, v_o):
    vfb = vf[...]
    ufb = uf[...]
    hv_o[...] = jnp.dot(vfb, Wiv[...], preferred_element_type=jnp.float32, precision=_PREC)
    us_o[...] = jnp.dot(ufb, Wiu[...], preferred_element_type=jnp.float32, precision=_PREC) + bi[...]
    u_o[...] = jnp.dot(ufb, Wu[...], preferred_element_type=jnp.float32, precision=_PREC) + bu[...]
    v_o[...] = jnp.dot(vfb, Wv[...], preferred_element_type=jnp.float32, precision=_PREC) + bv[...]


def _proj(v_feat, u_feat, Wiv, Wv, Wiu, Wu, bi, bu, bv):
    grid = (N // _BM,)
    return pl.pallas_call(
        _proj_body,
        grid=grid,
        in_specs=[
            pl.BlockSpec((_BM, D_V), lambda i: (i, 0)),
            pl.BlockSpec((_BM, D_U), lambda i: (i, 0)),
            pl.BlockSpec((D_V, D_OUT), lambda i: (0, 0)),
            pl.BlockSpec((D_V, D_EMB), lambda i: (0, 0)),
            pl.BlockSpec((D_U, D_OUT), lambda i: (0, 0)),
            pl.BlockSpec((D_U, D_EMB), lambda i: (0, 0)),
            pl.BlockSpec((1, D_OUT), lambda i: (0, 0)),
            pl.BlockSpec((1, D_EMB), lambda i: (0, 0)),
            pl.BlockSpec((1, D_EMB), lambda i: (0, 0)),
        ],
        out_specs=[
            pl.BlockSpec((_BM, D_OUT), lambda i: (i, 0)),
            pl.BlockSpec((_BM, D_OUT), lambda i: (i, 0)),
            pl.BlockSpec((_BM, D_EMB), lambda i: (i, 0)),
            pl.BlockSpec((_BM, D_EMB), lambda i: (i, 0)),
        ],
        out_shape=[
            jax.ShapeDtypeStruct((N, D_OUT), jnp.float32),
            jax.ShapeDtypeStruct((N, D_OUT), jnp.float32),
            jax.ShapeDtypeStruct((N, D_EMB), jnp.float32),
            jax.ShapeDtypeStruct((N, D_EMB), jnp.float32),
        ],
    )(v_feat, u_feat, Wiv, Wv, Wiu, Wu, bi, bu, bv)


def _gcn_body(a, h, st, W1, b1, W2, b2, o1, o2):
    ab = a[...]
    msg = jnp.dot(ab, h[...], preferred_element_type=jnp.float32, precision=_PREC)
    deg = jnp.sum(ab, axis=1, keepdims=True)
    x = jnp.maximum(msg / (deg + 1.0) + st[...], 0.0)
    o1[...] = jnp.dot(x, W1[...], preferred_element_type=jnp.float32, precision=_PREC) + b1[...]
    o2[...] = jnp.dot(x, W2[...], preferred_element_type=jnp.float32, precision=_PREC) + b2[...]


def _gcn(A, h, st, W1, b1, W2, b2):
    d1 = W1.shape[1]
    d2 = W2.shape[1]
    grid = (N // _BM,)
    return pl.pallas_call(
        _gcn_body,
        grid=grid,
        in_specs=[
            pl.BlockSpec((_BM, N), lambda i: (i, 0)),
            pl.BlockSpec((N, D_OUT), lambda i: (0, 0)),
            pl.BlockSpec((_BM, D_OUT), lambda i: (i, 0)),
            pl.BlockSpec((D_OUT, d1), lambda i: (0, 0)),
            pl.BlockSpec((1, d1), lambda i: (0, 0)),
            pl.BlockSpec((D_OUT, d2), lambda i: (0, 0)),
            pl.BlockSpec((1, d2), lambda i: (0, 0)),
        ],
        out_specs=[
            pl.BlockSpec((_BM, d1), lambda i: (i, 0)),
            pl.BlockSpec((_BM, d2), lambda i: (i, 0)),
        ],
        out_shape=[
            jax.ShapeDtypeStruct((N, d1), jnp.float32),
            jax.ShapeDtypeStruct((N, d2), jnp.float32),
        ],
    )(A, h, st, W1, b1, W2, b2)


_BQ = 512


def _attn_body(q, k, v, Wq, bq, Wk, bk, Wv, bv, Wo, bo, o):
    qh = (jnp.dot(q[...], Wq[...], preferred_element_type=jnp.float32, precision=_PREC) + bq[...]) * 0.125
    kh = jnp.dot(k[...], Wk[...], preferred_element_type=jnp.float32, precision=_PREC) + bk[...]
    vh = jnp.dot(v[...], Wv[...], preferred_element_type=jnp.float32, precision=_PREC) + bv[...]
    s = lax.dot_general(qh, kh, (((1,), (1,)), ((), ())),
                        preferred_element_type=jnp.float32, precision=_PREC)
    e = jnp.exp(jnp.minimum(s, 60.0))
    p = e * (1.0 / jnp.sum(e, axis=1, keepdims=True))
    o[...] = jnp.dot(jnp.dot(p, vh, preferred_element_type=jnp.float32, precision=_PREC),
                     Wo[...], preferred_element_type=jnp.float32, precision=_PREC) + bo[...]


def _attn(q, k, v, Wq, bq, Wk, bk, Wv, bv, Wo, bo):
    grid = (N // _BQ,)
    wspec = pl.BlockSpec((D_EMB, D_EMB), lambda i: (0, 0))
    bspec = pl.BlockSpec((1, D_EMB), lambda i: (0, 0))
    return pl.pallas_call(
        _attn_body,
        grid=grid,
        in_specs=[
            pl.BlockSpec((_BQ, D_EMB), lambda i: (i, 0)),
            pl.BlockSpec((N, D_EMB), lambda i: (0, 0)),
            pl.BlockSpec((N, D_EMB), lambda i: (0, 0)),
            wspec, bspec, wspec, bspec, wspec, bspec, wspec, bspec,
        ],
        out_specs=pl.BlockSpec((_BQ, D_EMB), lambda i: (i, 0)),
        out_shape=jax.ShapeDtypeStruct((N, D_EMB), jnp.float32),
    )(q, k, v, Wq, bq, Wk, bk, Wv, bv, Wo, bo)


def _gat_body(a, u2, v2, Wsrc, Wdst, asrc, adst, bg, o):
    ab = a[...]
    hsrc = jnp.dot(v2[...], Wsrc[...], preferred_element_type=jnp.float32, precision=_PREC)
    hdst = jnp.dot(u2[...], Wdst[...], preferred_element_type=jnp.float32, precision=_PREC)
    ssrc = lax.dot_general(asrc[...], hsrc, (((1,), (1,)), ((), ())),
                           preferred_element_type=jnp.float32, precision=_PREC)
    sdst = jnp.dot(hdst, adst[...], preferred_element_type=jnp.float32, precision=_PREC)
    z = sdst + ssrc
    e = jnp.where(z >= 0, z, 0.2 * z)
    mask = ab > 0.0
    m = jnp.max(jnp.where(mask, e, -1e30), axis=1, keepdims=True)
    w = ab * jnp.exp(jnp.minimum(e - m, 0.0))
    denom = jnp.sum(w, axis=1, keepdims=True)
    out = jnp.dot(w, hsrc, preferred_element_type=jnp.float32, precision=_PREC)
    o[...] = out / (denom + 1e-9) + bg[...]


def _gat(A, u2, v2, Wsrc, Wdst, asrc, adst, bg):
    grid = (N // _BM,)
    return pl.pallas_call(
        _gat_body,
        grid=grid,
        in_specs=[
            pl.BlockSpec((_BM, N), lambda i: (i, 0)),
            pl.BlockSpec((_BM, 2 * D_EMB), lambda i: (i, 0)),
            pl.BlockSpec((N, 2 * D_EMB), lambda i: (0, 0)),
            pl.BlockSpec((2 * D_EMB, D_EMB), lambda i: (0, 0)),
            pl.BlockSpec((2 * D_EMB, D_EMB), lambda i: (0, 0)),
            pl.BlockSpec((1, D_EMB), lambda i: (0, 0)),
            pl.BlockSpec((D_EMB, 1), lambda i: (0, 0)),
            pl.BlockSpec((1, D_EMB), lambda i: (0, 0)),
        ],
        out_specs=pl.BlockSpec((_BM, D_EMB), lambda i: (i, 0)),
        out_shape=jax.ShapeDtypeStruct((N, D_EMB), jnp.float32),
    )(A, u2, v2, Wsrc, Wdst, asrc, adst, bg)


def kernel(u_feat, v_feat, params, inter_src, inter_dst, social_src, social_dst):
    p = params
    isrc = inter_src.astype(jnp.int32)
    idst = inter_dst.astype(jnp.int32)
    ssrc = social_src.astype(jnp.int32)
    sdst = social_dst.astype(jnp.int32)

    A_inter, A_soc = _build_adj(isrc, idst, ssrc, sdst)

    row = lambda b: b.reshape(1, -1)
    h_v, u_self, u, v = _proj(
        v_feat, u_feat, p['W_inter_v'], p['W_v'], p['W_inter_u'], p['W_u'],
        row(p['b_inter']), row(p['b_u']), row(p['b_v']))

    zero_row = jnp.zeros((1, D_OUT), jnp.float32)
    h_soc, xs = _gcn(A_inter, h_v, u_self,
                     p['W_soc_nbr'], zero_row,
                     p['W_soc_self'], row(p['b_soc']))
    Xe, _ = _gcn(A_soc, h_soc, xs,
                 p['W_x'], row(p['b_x']),
                 p['W_x'], row(p['b_x']))

    def attn(q, k, v_, name):
        return _attn(q, k, v_,
                     p[name + '_Wq'], row(p[name + '_bq']),
                     p[name + '_Wk'], row(p[name + '_bk']),
                     p[name + '_Wv'], row(p[name + '_bv']),
                     p[name + '_Wo'], row(p[name + '_bo']))

    f_uus = attn(u, Xe, Xe, 'hur')
    e_uv = attn(v, f_uus, f_uus, 'uvr')
    e_vu = attn(f_uus, v, v, 'vur')

    u2 = jnp.concatenate([u, e_vu], axis=1)
    v2 = jnp.concatenate([v, e_uv], axis=1)

    return _gat(A_inter, u2, v2,
                p['W_gat_src'], p['W_gat_dst'],
                p['a_src'].reshape(1, D_EMB), p['a_dst'].reshape(D_EMB, 1),
                row(p['b_gat']))


# speedup vs baseline: 2.0307x; 1.0282x over previous
import functools

import jax
import jax.numpy as jnp
from jax import lax
from jax.experimental import pallas as pl
from jax.experimental.pallas import tpu as pltpu
from jax.experimental.pallas import tpu_sc as plsc

_PREC = lax.Precision.HIGHEST

N = 4096
D_U = 256
D_V = 2304
D_OUT = 512
D_EMB = 64
E = 65536

_NW = 32
_ROWS = 16
_PASSES = N // (_ROWS * _NW)
_CHUNK = 2048


def _adj_body(src_hbm, dst_hbm, zeros_hbm, a_hbm, ablk, src_v, dst_v):
    wid = lax.axis_index("s") * 2 + lax.axis_index("c")
    ones16 = jnp.ones((16,), jnp.float32)

    if True:
        def do_pass(p, carry):
            r0 = (p * _NW + wid) * _ROWS
            pltpu.sync_copy(zeros_hbm, ablk)

            def chunk_body(ci, c):
                pltpu.sync_copy(src_hbm.at[pl.ds(ci * _CHUNK, _CHUNK)], src_v)
                pltpu.sync_copy(dst_hbm.at[pl.ds(ci * _CHUNK, _CHUNK)], dst_v)

                @plsc.parallel_loop(0, _CHUNK, 16, unroll=8)
                def ebody(i):
                    s16 = src_v[pl.ds(i, 16)]
                    d16 = dst_v[pl.ds(i, 16)]
                    rel = d16 - r0
                    msk = plsc.bitcast(rel, jnp.uint32) < jnp.uint32(_ROWS)
                    plsc.addupdate_scatter(ablk, [rel * N + s16], ones16,
                                           mask=msk)
                return c
            lax.fori_loop(0, E // _CHUNK, chunk_body, 0)
            pltpu.sync_copy(ablk, a_hbm.at[pl.ds(r0 * N, _ROWS * N)])
            return carry
        lax.fori_loop(0, _PASSES, do_pass, 0)


def _build_adj_one(src, dst, zeros_blk):
    mesh = plsc.VectorSubcoreMesh(core_axis_name="c", subcore_axis_name="s")
    f = pl.kernel(
        _adj_body,
        out_type=jax.ShapeDtypeStruct((N * N,), jnp.float32),
        mesh=mesh,
        scratch_types=[
            pltpu.VMEM((_ROWS * N,), jnp.float32),
            pltpu.VMEM((_CHUNK,), jnp.int32),
            pltpu.VMEM((_CHUNK,), jnp.int32),
        ],
        compiler_params=pltpu.CompilerParams(needs_layout_passes=False),
    )
    return f(src, dst, zeros_blk).reshape(N, N)


def _build_adj(isrc, idst, ssrc, sdst):
    zeros_blk = jnp.zeros((_ROWS * N,), jnp.float32)
    ai = _build_adj_one(isrc, idst, zeros_blk)
    asoc = _build_adj_one(ssrc, sdst, zeros_blk)
    return ai, asoc


_BM = 256


def _proj_body(vf, uf, Wiv, Wv, Wiu, Wu, bi, bu, bv, hv_o, us_o, u_o, v_o):
    vfb = vf[...]
    ufb = uf[...]
    hv_o[...] = jnp.dot(vfb, Wiv[...], preferred_element_type=jnp.float32, precision=_PREC)
    us_o[...] = jnp.dot(ufb, Wiu[...], preferred_element_type=jnp.float32, precision=_PREC) + bi[...]
    u_o[...] = jnp.dot(ufb, Wu[...], preferred_element_type=jnp.float32, precision=_PREC) + bu[...]
    v_o[...] = jnp.dot(vfb, Wv[...], preferred_element_type=jnp.float32, precision=_PREC) + bv[...]


def _proj(v_feat, u_feat, Wiv, Wv, Wiu, Wu, bi, bu, bv):
    grid = (N // _BM,)
    return pl.pallas_call(
        _proj_body,
        grid=grid,
        in_specs=[
            pl.BlockSpec((_BM, D_V), lambda i: (i, 0)),
            pl.BlockSpec((_BM, D_U), lambda i: (i, 0)),
            pl.BlockSpec((D_V, D_OUT), lambda i: (0, 0)),
            pl.BlockSpec((D_V, D_EMB), lambda i: (0, 0)),
            pl.BlockSpec((D_U, D_OUT), lambda i: (0, 0)),
            pl.BlockSpec((D_U, D_EMB), lambda i: (0, 0)),
            pl.BlockSpec((1, D_OUT), lambda i: (0, 0)),
            pl.BlockSpec((1, D_EMB), lambda i: (0, 0)),
            pl.BlockSpec((1, D_EMB), lambda i: (0, 0)),
        ],
        out_specs=[
            pl.BlockSpec((_BM, D_OUT), lambda i: (i, 0)),
            pl.BlockSpec((_BM, D_OUT), lambda i: (i, 0)),
            pl.BlockSpec((_BM, D_EMB), lambda i: (i, 0)),
            pl.BlockSpec((_BM, D_EMB), lambda i: (i, 0)),
        ],
        out_shape=[
            jax.ShapeDtypeStruct((N, D_OUT), jnp.float32),
            jax.ShapeDtypeStruct((N, D_OUT), jnp.float32),
            jax.ShapeDtypeStruct((N, D_EMB), jnp.float32),
            jax.ShapeDtypeStruct((N, D_EMB), jnp.float32),
        ],
    )(v_feat, u_feat, Wiv, Wv, Wiu, Wu, bi, bu, bv)


def _gcn_body(a, hhi, hlo, st, W1, b1, W2, b2, o1, o2):
    ab = a[...]
    ab16 = ab.astype(jnp.bfloat16)
    msg = (jnp.dot(ab16, hhi[...], preferred_element_type=jnp.float32)
           + jnp.dot(ab16, hlo[...], preferred_element_type=jnp.float32))
    deg = jnp.sum(ab, axis=1, keepdims=True)
    x = jnp.maximum(msg / (deg + 1.0) + st[...], 0.0)
    o1[...] = jnp.dot(x, W1[...], preferred_element_type=jnp.float32, precision=_PREC) + b1[...]
    o2[...] = jnp.dot(x, W2[...], preferred_element_type=jnp.float32, precision=_PREC) + b2[...]


def _gcn(A, h, st, W1, b1, W2, b2):
    d1 = W1.shape[1]
    d2 = W2.shape[1]
    hhi = h.astype(jnp.bfloat16)
    hlo = (h - hhi.astype(jnp.float32)).astype(jnp.bfloat16)
    grid = (N // _BM,)
    return pl.pallas_call(
        _gcn_body,
        grid=grid,
        in_specs=[
            pl.BlockSpec((_BM, N), lambda i: (i, 0)),
            pl.BlockSpec((N, D_OUT), lambda i: (0, 0)),
            pl.BlockSpec((N, D_OUT), lambda i: (0, 0)),
            pl.BlockSpec((_BM, D_OUT), lambda i: (i, 0)),
            pl.BlockSpec((D_OUT, d1), lambda i: (0, 0)),
            pl.BlockSpec((1, d1), lambda i: (0, 0)),
            pl.BlockSpec((D_OUT, d2), lambda i: (0, 0)),
            pl.BlockSpec((1, d2), lambda i: (0, 0)),
        ],
        out_specs=[
            pl.BlockSpec((_BM, d1), lambda i: (i, 0)),
            pl.BlockSpec((_BM, d2), lambda i: (i, 0)),
        ],
        out_shape=[
            jax.ShapeDtypeStruct((N, d1), jnp.float32),
            jax.ShapeDtypeStruct((N, d2), jnp.float32),
        ],
    )(A, hhi, hlo, st, W1, b1, W2, b2)


_BQ = 512


def _attn_body(q, k, v, Wq, bq, Wk, bk, Wv, bv, Wo, bo, o):
    qh = (jnp.dot(q[...], Wq[...], preferred_element_type=jnp.float32, precision=_PREC) + bq[...]) * 0.125
    kh = jnp.dot(k[...], Wk[...], preferred_element_type=jnp.float32, precision=_PREC) + bk[...]
    vh = jnp.dot(v[...], Wv[...], preferred_element_type=jnp.float32, precision=_PREC) + bv[...]
    s = lax.dot_general(qh, kh, (((1,), (1,)), ((), ())),
                        preferred_element_type=jnp.float32, precision=_PREC)
    e = jnp.exp(jnp.minimum(s, 60.0))
    p = e * (1.0 / jnp.sum(e, axis=1, keepdims=True))
    o[...] = jnp.dot(jnp.dot(p, vh, preferred_element_type=jnp.float32, precision=_PREC),
                     Wo[...], preferred_element_type=jnp.float32, precision=_PREC) + bo[...]


def _attn(q, k, v, Wq, bq, Wk, bk, Wv, bv, Wo, bo):
    grid = (N // _BQ,)
    wspec = pl.BlockSpec((D_EMB, D_EMB), lambda i: (0, 0))
    bspec = pl.BlockSpec((1, D_EMB), lambda i: (0, 0))
    return pl.pallas_call(
        _attn_body,
        grid=grid,
        in_specs=[
            pl.BlockSpec((_BQ, D_EMB), lambda i: (i, 0)),
            pl.BlockSpec((N, D_EMB), lambda i: (0, 0)),
            pl.BlockSpec((N, D_EMB), lambda i: (0, 0)),
            wspec, bspec, wspec, bspec, wspec, bspec, wspec, bspec,
        ],
        out_specs=pl.BlockSpec((_BQ, D_EMB), lambda i: (i, 0)),
        out_shape=jax.ShapeDtypeStruct((N, D_EMB), jnp.float32),
    )(q, k, v, Wq, bq, Wk, bk, Wv, bv, Wo, bo)


def _gat_body(a, u2, v2, Wsrc, Wdst, asrc, adst, bg, o):
    ab = a[...]
    hsrc = jnp.dot(v2[...], Wsrc[...], preferred_element_type=jnp.float32, precision=_PREC)
    hdst = jnp.dot(u2[...], Wdst[...], preferred_element_type=jnp.float32, precision=_PREC)
    ssrc = lax.dot_general(asrc[...], hsrc, (((1,), (1,)), ((), ())),
                           preferred_element_type=jnp.float32, precision=_PREC)
    sdst = jnp.dot(hdst, adst[...], preferred_element_type=jnp.float32, precision=_PREC)
    z = sdst + ssrc
    e = jnp.where(z >= 0, z, 0.2 * z)
    mask = ab > 0.0
    m = jnp.max(jnp.where(mask, e, -1e30), axis=1, keepdims=True)
    w = ab * jnp.exp(jnp.minimum(e - m, 0.0))
    denom = jnp.sum(w, axis=1, keepdims=True)
    out = jnp.dot(w, hsrc, preferred_element_type=jnp.float32, precision=_PREC)
    o[...] = out / (denom + 1e-9) + bg[...]


def _gat(A, u2, v2, Wsrc, Wdst, asrc, adst, bg):
    grid = (N // _BM,)
    return pl.pallas_call(
        _gat_body,
        grid=grid,
        in_specs=[
            pl.BlockSpec((_BM, N), lambda i: (i, 0)),
            pl.BlockSpec((_BM, 2 * D_EMB), lambda i: (i, 0)),
            pl.BlockSpec((N, 2 * D_EMB), lambda i: (0, 0)),
            pl.BlockSpec((2 * D_EMB, D_EMB), lambda i: (0, 0)),
            pl.BlockSpec((2 * D_EMB, D_EMB), lambda i: (0, 0)),
            pl.BlockSpec((1, D_EMB), lambda i: (0, 0)),
            pl.BlockSpec((D_EMB, 1), lambda i: (0, 0)),
            pl.BlockSpec((1, D_EMB), lambda i: (0, 0)),
        ],
        out_specs=pl.BlockSpec((_BM, D_EMB), lambda i: (i, 0)),
        out_shape=jax.ShapeDtypeStruct((N, D_EMB), jnp.float32),
    )(A, u2, v2, Wsrc, Wdst, asrc, adst, bg)


def kernel(u_feat, v_feat, params, inter_src, inter_dst, social_src, social_dst):
    p = params
    isrc = inter_src.astype(jnp.int32)
    idst = inter_dst.astype(jnp.int32)
    ssrc = social_src.astype(jnp.int32)
    sdst = social_dst.astype(jnp.int32)

    A_inter, A_soc = _build_adj(isrc, idst, ssrc, sdst)

    row = lambda b: b.reshape(1, -1)
    h_v, u_self, u, v = _proj(
        v_feat, u_feat, p['W_inter_v'], p['W_v'], p['W_inter_u'], p['W_u'],
        row(p['b_inter']), row(p['b_u']), row(p['b_v']))

    zero_row = jnp.zeros((1, D_OUT), jnp.float32)
    h_soc, xs = _gcn(A_inter, h_v, u_self,
                     p['W_soc_nbr'], zero_row,
                     p['W_soc_self'], row(p['b_soc']))
    Xe, _ = _gcn(A_soc, h_soc, xs,
                 p['W_x'], row(p['b_x']),
                 p['W_x'], row(p['b_x']))

    def attn(q, k, v_, name):
        return _attn(q, k, v_,
                     p[name + '_Wq'], row(p[name + '_bq']),
                     p[name + '_Wk'], row(p[name + '_bk']),
                     p[name + '_Wv'], row(p[name + '_bv']),
                     p[name + '_Wo'], row(p[name + '_bo']))

    f_uus = attn(u, Xe, Xe, 'hur')
    e_uv = attn(v, f_uus, f_uus, 'uvr')
    e_vu = attn(f_uus, v, v, 'vur')

    u2 = jnp.concatenate([u, e_vu], axis=1)
    v2 = jnp.concatenate([v, e_uv], axis=1)

    return _gat(A_inter, u2, v2,
                p['W_gat_src'], p['W_gat_dst'],
                p['a_src'].reshape(1, D_EMB), p['a_dst'].reshape(D_EMB, 1),
                row(p['b_gat']))


# speedup vs baseline: 2.5331x; 1.2474x over previous
import functools

import jax
import jax.numpy as jnp
from jax import lax
from jax.experimental import pallas as pl
from jax.experimental.pallas import tpu as pltpu
from jax.experimental.pallas import tpu_sc as plsc

_PREC = lax.Precision.HIGHEST

N = 4096
D_U = 256
D_V = 2304
D_OUT = 512
D_EMB = 64
E = 65536

_NW = 32
_ROWS = 16
_PASSES = N // (_ROWS * _NW)
_CHUNK = 8192


def _adj_body(src_hbm, dst_hbm, zeros_hbm, a_hbm, ablk, src_v, dst_v):
    wid = lax.axis_index("s") * 2 + lax.axis_index("c")
    ones16 = jnp.ones((16,), jnp.float32)

    if True:
        def do_pass(p, carry):
            r0 = (p * _NW + wid) * _ROWS
            pltpu.sync_copy(zeros_hbm, ablk)

            def chunk_body(ci, c):
                pltpu.sync_copy(src_hbm.at[pl.ds(ci * _CHUNK, _CHUNK)], src_v)
                pltpu.sync_copy(dst_hbm.at[pl.ds(ci * _CHUNK, _CHUNK)], dst_v)

                @plsc.parallel_loop(0, _CHUNK, 16, unroll=16)
                def ebody(i):
                    s16 = src_v[pl.ds(i, 16)]
                    d16 = dst_v[pl.ds(i, 16)]
                    rel = d16 - r0
                    msk = plsc.bitcast(rel, jnp.uint32) < jnp.uint32(_ROWS)
                    plsc.addupdate_scatter(ablk, [rel * N + s16], ones16,
                                           mask=msk)
                return c
            lax.fori_loop(0, E // _CHUNK, chunk_body, 0)
            pltpu.sync_copy(ablk, a_hbm.at[pl.ds(r0 * N, _ROWS * N)])
            return carry
        lax.fori_loop(0, _PASSES, do_pass, 0)


def _build_adj_one(src, dst, zeros_blk):
    mesh = plsc.VectorSubcoreMesh(core_axis_name="c", subcore_axis_name="s")
    f = pl.kernel(
        _adj_body,
        out_type=jax.ShapeDtypeStruct((N * N,), jnp.float32),
        mesh=mesh,
        scratch_types=[
            pltpu.VMEM((_ROWS * N,), jnp.float32),
            pltpu.VMEM((_CHUNK,), jnp.int32),
            pltpu.VMEM((_CHUNK,), jnp.int32),
        ],
        compiler_params=pltpu.CompilerParams(needs_layout_passes=False),
    )
    return f(src, dst, zeros_blk).reshape(N, N)


def _build_adj(isrc, idst, ssrc, sdst):
    zeros_blk = jnp.zeros((_ROWS * N,), jnp.float32)
    ai = _build_adj_one(isrc, idst, zeros_blk)
    asoc = _build_adj_one(ssrc, sdst, zeros_blk)
    return ai, asoc


_BM = 256


def _proj_body(vf, uf, Wiv, Wv, Wiu, Wu, bi, bu, bv, hv_o, us_o, u_o, v_o):
    vfb = vf[...]
    ufb = uf[...]
    hv_o[...] = jnp.dot(vfb, Wiv[...], preferred_element_type=jnp.float32, precision=_PREC)
    us_o[...] = jnp.dot(ufb, Wiu[...], preferred_element_type=jnp.float32, precision=_PREC) + bi[...]
    u_o[...] = jnp.dot(ufb, Wu[...], preferred_element_type=jnp.float32, precision=_PREC) + bu[...]
    v_o[...] = jnp.dot(vfb, Wv[...], preferred_element_type=jnp.float32, precision=_PREC) + bv[...]


def _proj(v_feat, u_feat, Wiv, Wv, Wiu, Wu, bi, bu, bv):
    grid = (N // _BM,)
    return pl.pallas_call(
        _proj_body,
        grid=grid,
        in_specs=[
            pl.BlockSpec((_BM, D_V), lambda i: (i, 0)),
            pl.BlockSpec((_BM, D_U), lambda i: (i, 0)),
            pl.BlockSpec((D_V, D_OUT), lambda i: (0, 0)),
            pl.BlockSpec((D_V, D_EMB), lambda i: (0, 0)),
            pl.BlockSpec((D_U, D_OUT), lambda i: (0, 0)),
            pl.BlockSpec((D_U, D_EMB), lambda i: (0, 0)),
            pl.BlockSpec((1, D_OUT), lambda i: (0, 0)),
            pl.BlockSpec((1, D_EMB), lambda i: (0, 0)),
            pl.BlockSpec((1, D_EMB), lambda i: (0, 0)),
        ],
        out_specs=[
            pl.BlockSpec((_BM, D_OUT), lambda i: (i, 0)),
            pl.BlockSpec((_BM, D_OUT), lambda i: (i, 0)),
            pl.BlockSpec((_BM, D_EMB), lambda i: (i, 0)),
            pl.BlockSpec((_BM, D_EMB), lambda i: (i, 0)),
        ],
        out_shape=[
            jax.ShapeDtypeStruct((N, D_OUT), jnp.float32),
            jax.ShapeDtypeStruct((N, D_OUT), jnp.float32),
            jax.ShapeDtypeStruct((N, D_EMB), jnp.float32),
            jax.ShapeDtypeStruct((N, D_EMB), jnp.float32),
        ],
    )(v_feat, u_feat, Wiv, Wv, Wiu, Wu, bi, bu, bv)


def _gcn_body(a, hhi, hlo, st, W1, b1, W2, b2, o1, o2):
    ab = a[...]
    ab16 = ab.astype(jnp.bfloat16)
    msg = (jnp.dot(ab16, hhi[...], preferred_element_type=jnp.float32)
           + jnp.dot(ab16, hlo[...], preferred_element_type=jnp.float32))
    deg = jnp.sum(ab, axis=1, keepdims=True)
    x = jnp.maximum(msg / (deg + 1.0) + st[...], 0.0)
    o1[...] = jnp.dot(x, W1[...], preferred_element_type=jnp.float32, precision=_PREC) + b1[...]
    o2[...] = jnp.dot(x, W2[...], preferred_element_type=jnp.float32, precision=_PREC) + b2[...]


def _gcn(A, h, st, W1, b1, W2, b2):
    d1 = W1.shape[1]
    d2 = W2.shape[1]
    hhi = h.astype(jnp.bfloat16)
    hlo = (h - hhi.astype(jnp.float32)).astype(jnp.bfloat16)
    grid = (N // _BM,)
    return pl.pallas_call(
        _gcn_body,
        grid=grid,
        in_specs=[
            pl.BlockSpec((_BM, N), lambda i: (i, 0)),
            pl.BlockSpec((N, D_OUT), lambda i: (0, 0)),
            pl.BlockSpec((N, D_OUT), lambda i: (0, 0)),
            pl.BlockSpec((_BM, D_OUT), lambda i: (i, 0)),
            pl.BlockSpec((D_OUT, d1), lambda i: (0, 0)),
            pl.BlockSpec((1, d1), lambda i: (0, 0)),
            pl.BlockSpec((D_OUT, d2), lambda i: (0, 0)),
            pl.BlockSpec((1, d2), lambda i: (0, 0)),
        ],
        out_specs=[
            pl.BlockSpec((_BM, d1), lambda i: (i, 0)),
            pl.BlockSpec((_BM, d2), lambda i: (i, 0)),
        ],
        out_shape=[
            jax.ShapeDtypeStruct((N, d1), jnp.float32),
            jax.ShapeDtypeStruct((N, d2), jnp.float32),
        ],
    )(A, hhi, hlo, st, W1, b1, W2, b2)


_BQ = 512


def _attn_body(q, k, v, Wq, bq, Wk, bk, Wv, bv, Wo, bo, o):
    qh = (jnp.dot(q[...], Wq[...], preferred_element_type=jnp.float32, precision=_PREC) + bq[...]) * 0.125
    kh = jnp.dot(k[...], Wk[...], preferred_element_type=jnp.float32, precision=_PREC) + bk[...]
    vh = jnp.dot(v[...], Wv[...], preferred_element_type=jnp.float32, precision=_PREC) + bv[...]
    s = lax.dot_general(qh, kh, (((1,), (1,)), ((), ())),
                        preferred_element_type=jnp.float32, precision=_PREC)
    e = jnp.exp(jnp.minimum(s, 60.0))
    p = e * (1.0 / jnp.sum(e, axis=1, keepdims=True))
    o[...] = jnp.dot(jnp.dot(p, vh, preferred_element_type=jnp.float32, precision=_PREC),
                     Wo[...], preferred_element_type=jnp.float32, precision=_PREC) + bo[...]


def _attn(q, k, v, Wq, bq, Wk, bk, Wv, bv, Wo, bo):
    grid = (N // _BQ,)
    wspec = pl.BlockSpec((D_EMB, D_EMB), lambda i: (0, 0))
    bspec = pl.BlockSpec((1, D_EMB), lambda i: (0, 0))
    return pl.pallas_call(
        _attn_body,
        grid=grid,
        in_specs=[
            pl.BlockSpec((_BQ, D_EMB), lambda i: (i, 0)),
            pl.BlockSpec((N, D_EMB), lambda i: (0, 0)),
            pl.BlockSpec((N, D_EMB), lambda i: (0, 0)),
            wspec, bspec, wspec, bspec, wspec, bspec, wspec, bspec,
        ],
        out_specs=pl.BlockSpec((_BQ, D_EMB), lambda i: (i, 0)),
        out_shape=jax.ShapeDtypeStruct((N, D_EMB), jnp.float32),
    )(q, k, v, Wq, bq, Wk, bk, Wv, bv, Wo, bo)


def _gat_body(a, u2, v2, Wsrc, Wdst, asrc, adst, bg, o):
    ab = a[...]
    hsrc = jnp.dot(v2[...], Wsrc[...], preferred_element_type=jnp.float32, precision=_PREC)
    hdst = jnp.dot(u2[...], Wdst[...], preferred_element_type=jnp.float32, precision=_PREC)
    ssrc = lax.dot_general(asrc[...], hsrc, (((1,), (1,)), ((), ())),
                           preferred_element_type=jnp.float32, precision=_PREC)
    sdst = jnp.dot(hdst, adst[...], preferred_element_type=jnp.float32, precision=_PREC)
    z = sdst + ssrc
    e = jnp.where(z >= 0, z, 0.2 * z)
    mask = ab > 0.0
    m = jnp.max(jnp.where(mask, e, -1e30), axis=1, keepdims=True)
    w = ab * jnp.exp(jnp.minimum(e - m, 0.0))
    denom = jnp.sum(w, axis=1, keepdims=True)
    out = jnp.dot(w, hsrc, preferred_element_type=jnp.float32, precision=_PREC)
    o[...] = out / (denom + 1e-9) + bg[...]


def _gat(A, u2, v2, Wsrc, Wdst, asrc, adst, bg):
    grid = (N // _BM,)
    return pl.pallas_call(
        _gat_body,
        grid=grid,
        in_specs=[
            pl.BlockSpec((_BM, N), lambda i: (i, 0)),
            pl.BlockSpec((_BM, 2 * D_EMB), lambda i: (i, 0)),
            pl.BlockSpec((N, 2 * D_EMB), lambda i: (0, 0)),
            pl.BlockSpec((2 * D_EMB, D_EMB), lambda i: (0, 0)),
            pl.BlockSpec((2 * D_EMB, D_EMB), lambda i: (0, 0)),
            pl.BlockSpec((1, D_EMB), lambda i: (0, 0)),
            pl.BlockSpec((D_EMB, 1), lambda i: (0, 0)),
            pl.BlockSpec((1, D_EMB), lambda i: (0, 0)),
        ],
        out_specs=pl.BlockSpec((_BM, D_EMB), lambda i: (i, 0)),
        out_shape=jax.ShapeDtypeStruct((N, D_EMB), jnp.float32),
    )(A, u2, v2, Wsrc, Wdst, asrc, adst, bg)


def kernel(u_feat, v_feat, params, inter_src, inter_dst, social_src, social_dst):
    p = params
    isrc = inter_src.astype(jnp.int32)
    idst = inter_dst.astype(jnp.int32)
    ssrc = social_src.astype(jnp.int32)
    sdst = social_dst.astype(jnp.int32)

    A_inter, A_soc = _build_adj(isrc, idst, ssrc, sdst)

    row = lambda b: b.reshape(1, -1)
    h_v, u_self, u, v = _proj(
        v_feat, u_feat, p['W_inter_v'], p['W_v'], p['W_inter_u'], p['W_u'],
        row(p['b_inter']), row(p['b_u']), row(p['b_v']))

    zero_row = jnp.zeros((1, D_OUT), jnp.float32)
    h_soc, xs = _gcn(A_inter, h_v, u_self,
                     p['W_soc_nbr'], zero_row,
                     p['W_soc_self'], row(p['b_soc']))
    Xe, _ = _gcn(A_soc, h_soc, xs,
                 p['W_x'], row(p['b_x']),
                 p['W_x'], row(p['b_x']))

    def attn(q, k, v_, name):
        return _attn(q, k, v_,
                     p[name + '_Wq'], row(p[name + '_bq']),
                     p[name + '_Wk'], row(p[name + '_bk']),
                     p[name + '_Wv'], row(p[name + '_bv']),
                     p[name + '_Wo'], row(p[name + '_bo']))

    f_uus = attn(u, Xe, Xe, 'hur')
    e_uv = attn(v, f_uus, f_uus, 'uvr')
    e_vu = attn(f_uus, v, v, 'vur')

    u2 = jnp.concatenate([u, e_vu], axis=1)
    v2 = jnp.concatenate([v, e_uv], axis=1)

    return _gat(A_inter, u2, v2,
                p['W_gat_src'], p['W_gat_dst'],
                p['a_src'].reshape(1, D_EMB), p['a_dst'].reshape(D_EMB, 1),
                row(p['b_gat']))
